# baseline (speedup 1.0000x reference)
import functools

import jax
import jax.numpy as jnp
from jax import lax
from jax.experimental import pallas as pl
from jax.experimental.pallas import tpu as pltpu
from jax.experimental.pallas import tpu_sc as plsc

D = 256
K = 8192
N = 8192
NB = 8
KB = 8
BN = N // NB
BK = K // KB

_SC_NC = 2
_SC_NS = 16
_SC_NW = _SC_NC * _SC_NS
_B_PER_W = N // _SC_NW


def _vq_body(xt_ref, w_ref, idx_ref, enc_ref, loss_ref, perp_ref,
             mv_s, mi_s, mi_col, cnt_s, acc_s):
    nb = pl.program_id(0)
    kb = pl.program_id(1)

    @pl.when(kb < KB)
    def _():
        xt = xt_ref[...]
        w = w_ref[...]
        s = lax.dot_general(w, xt, (((1,), (0,)), ((), ())),
                            preferred_element_type=jnp.float32)
        s1 = jnp.sum(xt * xt, axis=0, keepdims=True)
        s2 = jnp.sum(w * w, axis=1, keepdims=True)
        d = (s1 - 2.0 * s) + s2
        bmin = jnp.min(d, axis=0, keepdims=True)
        idsf = lax.broadcasted_iota(jnp.int32, (BK, BN), 0).astype(jnp.float32)
        bidx = (jnp.min(jnp.where(d == bmin, idsf, jnp.float32(3e10)),
                        axis=0, keepdims=True)
                + (kb * BK).astype(jnp.float32))

        @pl.when(kb == 0)
        def _():
            mv_s[...] = bmin
            mi_s[...] = bidx

        @pl.when(kb > 0)
        def _():
            better = bmin < mv_s[...]
            mi_s[...] = jnp.where(better, bidx, mi_s[...])
            mv_s[...] = jnp.where(better, bmin, mv_s[...])

        @pl.when(kb == KB - 1)
        def _():
            mi = mi_s[...].astype(jnp.int32)
            idx_ref[...] = mi.reshape(BN)
            mi_col[...] = lax.transpose(mi, (1, 0))
            rowsum = jnp.sum(mv_s[...])

            @pl.when(nb == 0)
            def _():
                acc_s[0, 0] = rowsum

            @pl.when(nb > 0)
            def _():
                acc_s[0, 0] = acc_s[0, 0] + rowsum

    @pl.when(kb >= KB)
    def _():
        kbb = kb - KB
        idc = mi_col[...] - kbb * BK
        ids = lax.broadcasted_iota(jnp.int32, (BN, BK), 1)
        onehot = (ids == idc).astype(jnp.float32)
        enc_ref[...] = onehot
        cnt = jnp.sum(onehot, axis=0, keepdims=True)

        @pl.when(nb == 0)
        def _():
            cnt_s[:, pl.ds(kbb * BK, BK)] = cnt

        @pl.when(nb > 0)
        def _():
            cnt_s[:, pl.ds(kbb * BK, BK)] = (
                cnt_s[:, pl.ds(kbb * BK, BK)] + cnt)

        @pl.when(jnp.logical_and(nb == NB - 1, kb == 2 * KB - 1))
        def _():
            loss = 0.25 * acc_s[0, 0] * (1.0 / (N * D))
            loss_ref[...] = jnp.full((1, 128), loss, jnp.float32)
            p = cnt_s[...] * (1.0 / N)
            ent = jnp.sum(p * jnp.log(p + 1e-10))
            perp_ref[...] = jnp.exp(jnp.full((1, 128), -ent, jnp.float32))


def _sc_gather(table_hbm, idx_hbm, out_hbm, idx_v, rows_v, sem):
    wid = lax.axis_index("s") * _SC_NC + lax.axis_index("c")
    base = wid * _B_PER_W
    pltpu.sync_copy(idx_hbm.at[pl.ds(base, _B_PER_W)], idx_v)
    pltpu.async_copy(table_hbm.at[idx_v], rows_v, sem).wait()
    pltpu.sync_copy(rows_v, out_hbm.at[pl.ds(base, _B_PER_W)])


def _vq_call(xt, w):
    return pl.pallas_call(
        _vq_body,
        grid=(NB, 2 * KB),
        in_specs=[
            pl.BlockSpec((D, BN), lambda nb, kb: (0, nb)),
            pl.BlockSpec((BK, D), lambda nb, kb: (jnp.minimum(kb, KB - 1), 0)),
        ],
        out_specs=[
            pl.BlockSpec((BN,), lambda nb, kb: (nb,)),
            pl.BlockSpec((BN, BK),
                         lambda nb, kb: (nb, jnp.maximum(kb - KB, 0))),
            pl.BlockSpec((1, 128), lambda nb, kb: (0, 0)),
            pl.BlockSpec((1, 128), lambda nb, kb: (0, 0)),
        ],
        out_shape=[
            jax.ShapeDtypeStruct((N,), jnp.int32),
            jax.ShapeDtypeStruct((N, K), jnp.float32),
            jax.ShapeDtypeStruct((1, 128), jnp.float32),
            jax.ShapeDtypeStruct((1, 128), jnp.float32),
        ],
        scratch_shapes=[
            pltpu.VMEM((1, BN), jnp.float32),
            pltpu.VMEM((1, BN), jnp.float32),
            pltpu.VMEM((BN, 1), jnp.int32),
            pltpu.VMEM((1, K), jnp.float32),
            pltpu.SMEM((1, 1), jnp.float32),
        ],
    )(xt, w)


def kernel(inputTensor, emb_weights, ema_w, ema_cluster_size):
    del ema_w, ema_cluster_size
    flat = inputTensor.reshape(-1, D)
    xt = flat.T

    idx_flat, enc, loss_out, perp_out = _vq_call(xt, emb_weights)

    sc_gather = functools.partial(
        pl.kernel,
        mesh=plsc.VectorSubcoreMesh(core_axis_name="c", subcore_axis_name="s"),
        out_type=jax.ShapeDtypeStruct((N, D), jnp.float32),
        scratch_types=[
            pltpu.VMEM((_B_PER_W,), jnp.int32),
            pltpu.VMEM((_B_PER_W, D), jnp.float32),
            pltpu.SemaphoreType.DMA,
        ],
    )(_sc_gather)
    quantized = sc_gather(emb_weights, idx_flat)

    loss = loss_out[0, 0]
    perplexity = perp_out[0, 0]
    quantized_st = quantized.reshape(inputTensor.shape)
    return (loss, quantized_st, perplexity, enc)


# speedup vs baseline: 1.0553x; 1.0553x over previous
import functools

import jax
import jax.numpy as jnp
from jax import lax
from jax.experimental import pallas as pl
from jax.experimental.pallas import tpu as pltpu
from jax.experimental.pallas import tpu_sc as plsc

D = 256
K = 8192
N = 8192
NB = 8
KB = 8
BN = N // NB
BK = K // KB

_SC_NC = 2
_SC_NS = 16
_SC_NW = _SC_NC * _SC_NS
_B_PER_W = N // _SC_NW


def _argmin_body(xt_ref, w_ref, idx_ref, loss_ref, mv_s, mi_s, acc_s):
    nb = pl.program_id(0)
    kb = pl.program_id(1)
    xt = xt_ref[...]
    w = w_ref[...]
    s = lax.dot_general(w, xt, (((1,), (0,)), ((), ())),
                        preferred_element_type=jnp.float32)
    s1 = jnp.sum(xt * xt, axis=0, keepdims=True)
    s2 = jnp.sum(w * w, axis=1, keepdims=True)
    d = (s1 - 2.0 * s) + s2
    bmin = jnp.min(d, axis=0, keepdims=True)
    idsf = lax.broadcasted_iota(jnp.int32, (BK, BN), 0).astype(jnp.float32)
    bidx = (jnp.min(jnp.where(d == bmin, idsf, jnp.float32(3e10)),
                    axis=0, keepdims=True)
            + (kb * BK).astype(jnp.float32))

    @pl.when(kb == 0)
    def _():
        mv_s[...] = bmin
        mi_s[...] = bidx

    @pl.when(kb > 0)
    def _():
        better = bmin < mv_s[...]
        mi_s[...] = jnp.where(better, bidx, mi_s[...])
        mv_s[...] = jnp.where(better, bmin, mv_s[...])

    @pl.when(kb == KB - 1)
    def _():
        idx_ref[...] = mi_s[...].astype(jnp.int32).reshape(BN)
        rowsum = jnp.sum(mv_s[...])

        @pl.when(nb == 0)
        def _():
            acc_s[0, 0] = rowsum

        @pl.when(nb > 0)
        def _():
            acc_s[0, 0] = acc_s[0, 0] + rowsum

        @pl.when(nb == NB - 1)
        def _():
            loss = 0.25 * acc_s[0, 0] * (1.0 / (N * D))
            loss_ref[...] = jnp.full((1, 128), loss, jnp.float32)


def _onehot_body(idx_ref, enc_ref, perp_ref, cnt_s, ent_s):
    kb = pl.program_id(0)
    nb = pl.program_id(1)
    idx_row = idx_ref[...].reshape(1, BN)
    idx_col = lax.transpose(idx_row, (1, 0))
    ids = lax.broadcasted_iota(jnp.int32, (BN, BK), 1) + kb * BK
    onehot = (ids == idx_col).astype(jnp.float32)
    enc_ref[...] = onehot
    cnt = jnp.sum(onehot, axis=0, keepdims=True)

    @pl.when(nb == 0)
    def _():
        cnt_s[...] = cnt

    @pl.when(nb > 0)
    def _():
        cnt_s[...] = cnt_s[...] + cnt

    @pl.when(nb == NB - 1)
    def _():
        p = cnt_s[...] * (1.0 / N)
        ev = jnp.sum(p * jnp.log(p + 1e-10))

        @pl.when(kb == 0)
        def _():
            ent_s[0, 0] = ev

        @pl.when(kb > 0)
        def _():
            ent_s[0, 0] = ent_s[0, 0] + ev

        @pl.when(kb == KB - 1)
        def _():
            perp_ref[...] = jnp.exp(jnp.full((1, 128), -ent_s[0, 0],
                                             jnp.float32))


def _sc_gather(table_hbm, idx_hbm, out_hbm, idx_v, rows_v, sem):
    wid = lax.axis_index("s") * _SC_NC + lax.axis_index("c")
    base = wid * _B_PER_W
    pltpu.sync_copy(idx_hbm.at[pl.ds(base, _B_PER_W)], idx_v)
    pltpu.async_copy(table_hbm.at[idx_v], rows_v, sem).wait()
    pltpu.sync_copy(rows_v, out_hbm.at[pl.ds(base, _B_PER_W)])


def _argmin_call(xt, w):
    return pl.pallas_call(
        _argmin_body,
        grid=(NB, KB),
        in_specs=[
            pl.BlockSpec((D, BN), lambda nb, kb: (0, nb)),
            pl.BlockSpec((BK, D), lambda nb, kb: (kb, 0)),
        ],
        out_specs=[
            pl.BlockSpec((BN,), lambda nb, kb: (nb,)),
            pl.BlockSpec((1, 128), lambda nb, kb: (0, 0)),
        ],
        out_shape=[
            jax.ShapeDtypeStruct((N,), jnp.int32),
            jax.ShapeDtypeStruct((1, 128), jnp.float32),
        ],
        scratch_shapes=[
            pltpu.VMEM((1, BN), jnp.float32),
            pltpu.VMEM((1, BN), jnp.float32),
            pltpu.SMEM((1, 1), jnp.float32),
        ],
    )(xt, w)


def _onehot_call(idx_flat):
    return pl.pallas_call(
        _onehot_body,
        grid=(KB, NB),
        in_specs=[
            pl.BlockSpec((BN,), lambda kb, nb: (nb,)),
        ],
        out_specs=[
            pl.BlockSpec((BN, BK), lambda kb, nb: (nb, kb)),
            pl.BlockSpec((1, 128), lambda kb, nb: (0, 0)),
        ],
        out_shape=[
            jax.ShapeDtypeStruct((N, K), jnp.float32),
            jax.ShapeDtypeStruct((1, 128), jnp.float32),
        ],
        scratch_shapes=[
            pltpu.VMEM((1, BK), jnp.float32),
            pltpu.SMEM((1, 1), jnp.float32),
        ],
    )(idx_flat)


def kernel(inputTensor, emb_weights, ema_w, ema_cluster_size):
    del ema_w, ema_cluster_size
    flat = inputTensor.reshape(-1, D)
    xt = flat.T

    idx_flat, loss_out = _argmin_call(xt, emb_weights)

    sc_gather = functools.partial(
        pl.kernel,
        mesh=plsc.VectorSubcoreMesh(core_axis_name="c", subcore_axis_name="s"),
        out_type=jax.ShapeDtypeStruct((N, D), jnp.float32),
        scratch_types=[
            pltpu.VMEM((_B_PER_W,), jnp.int32),
            pltpu.VMEM((_B_PER_W, D), jnp.float32),
            pltpu.SemaphoreType.DMA,
        ],
    )(_sc_gather)
    quantized = sc_gather(emb_weights, idx_flat)

    enc, perp_out = _onehot_call(idx_flat)

    loss = loss_out[0, 0]
    perplexity = perp_out[0, 0]
    quantized_st = quantized.reshape(inputTensor.shape)
    return (loss, quantized_st, perplexity, enc)


# speedup vs baseline: 1.1066x; 1.0486x over previous
import functools

import jax
import jax.numpy as jnp
from jax import lax
from jax.experimental import pallas as pl
from jax.experimental.pallas import tpu as pltpu
from jax.experimental.pallas import tpu_sc as plsc

D = 256
K = 8192
N = 8192
NB = 2
KB = 8
BN = N // NB
BK = K // KB

_SC_NC = 2
_SC_NS = 16
_SC_NW = _SC_NC * _SC_NS
_B_PER_W = N // _SC_NW


def _argmin_body(xt_ref, w_ref, idx_ref, loss_ref, mv_s, mi_s, acc_s):
    nb = pl.program_id(0)
    kb = pl.program_id(1)
    xt = xt_ref[...]
    w = w_ref[...]
    s = lax.dot_general(w, xt, (((1,), (0,)), ((), ())),
                        preferred_element_type=jnp.float32)
    s1 = jnp.sum(xt * xt, axis=0, keepdims=True)
    s2 = jnp.sum(w * w, axis=1, keepdims=True)
    d = (s1 - 2.0 * s) + s2
    bmin = jnp.min(d, axis=0, keepdims=True)
    idsf = lax.broadcasted_iota(jnp.int32, (BK, BN), 0).astype(jnp.float32)
    bidx = (jnp.min(jnp.where(d == bmin, idsf, jnp.float32(3e10)),
                    axis=0, keepdims=True)
            + (kb * BK).astype(jnp.float32))

    @pl.when(kb == 0)
    def _():
        mv_s[...] = bmin
        mi_s[...] = bidx

    @pl.when(kb > 0)
    def _():
        better = bmin < mv_s[...]
        mi_s[...] = jnp.where(better, bidx, mi_s[...])
        mv_s[...] = jnp.where(better, bmin, mv_s[...])

    @pl.when(kb == KB - 1)
    def _():
        idx_ref[...] = mi_s[...].astype(jnp.int32).reshape(BN)
        rowsum = jnp.sum(mv_s[...])

        @pl.when(nb == 0)
        def _():
            acc_s[0, 0] = rowsum

        @pl.when(nb > 0)
        def _():
            acc_s[0, 0] = acc_s[0, 0] + rowsum

        @pl.when(nb == NB - 1)
        def _():
            loss = 0.25 * acc_s[0, 0] * (1.0 / (N * D))
            loss_ref[...] = jnp.full((1, 128), loss, jnp.float32)


def _onehot_body(idx_ref, enc_ref, perp_ref, cnt_s, ent_s):
    kb = pl.program_id(0)
    nb = pl.program_id(1)
    idx_row = idx_ref[...].reshape(1, BN)
    idx_col = lax.transpose(idx_row, (1, 0))
    ids = lax.broadcasted_iota(jnp.int32, (BN, BK), 1) + kb * BK
    onehot = (ids == idx_col).astype(jnp.float32)
    enc_ref[...] = onehot
    cnt = jnp.sum(onehot, axis=0, keepdims=True)

    @pl.when(nb == 0)
    def _():
        cnt_s[...] = cnt

    @pl.when(nb > 0)
    def _():
        cnt_s[...] = cnt_s[...] + cnt

    @pl.when(nb == NB - 1)
    def _():
        p = cnt_s[...] * (1.0 / N)
        ev = jnp.sum(p * jnp.log(p + 1e-10))

        @pl.when(kb == 0)
        def _():
            ent_s[0, 0] = ev

        @pl.when(kb > 0)
        def _():
            ent_s[0, 0] = ent_s[0, 0] + ev

        @pl.when(kb == KB - 1)
        def _():
            perp_ref[...] = jnp.exp(jnp.full((1, 128), -ent_s[0, 0],
                                             jnp.float32))


def _sc_gather(table_hbm, idx_hbm, out_hbm, idx_v, rows_v, sem):
    wid = lax.axis_index("s") * _SC_NC + lax.axis_index("c")
    base = wid * _B_PER_W
    pltpu.sync_copy(idx_hbm.at[pl.ds(base, _B_PER_W)], idx_v)
    pltpu.async_copy(table_hbm.at[idx_v], rows_v, sem).wait()
    pltpu.sync_copy(rows_v, out_hbm.at[pl.ds(base, _B_PER_W)])


def _argmin_call(xt, w):
    return pl.pallas_call(
        _argmin_body,
        grid=(NB, KB),
        in_specs=[
            pl.BlockSpec((D, BN), lambda nb, kb: (0, nb)),
            pl.BlockSpec((BK, D), lambda nb, kb: (kb, 0)),
        ],
        out_specs=[
            pl.BlockSpec((BN,), lambda nb, kb: (nb,)),
            pl.BlockSpec((1, 128), lambda nb, kb: (0, 0)),
        ],
        out_shape=[
            jax.ShapeDtypeStruct((N,), jnp.int32),
            jax.ShapeDtypeStruct((1, 128), jnp.float32),
        ],
        scratch_shapes=[
            pltpu.VMEM((1, BN), jnp.float32),
            pltpu.VMEM((1, BN), jnp.float32),
            pltpu.SMEM((1, 1), jnp.float32),
        ],
    )(xt, w)


def _onehot_call(idx_flat):
    return pl.pallas_call(
        _onehot_body,
        grid=(KB, NB),
        in_specs=[
            pl.BlockSpec((BN,), lambda kb, nb: (nb,)),
        ],
        out_specs=[
            pl.BlockSpec((BN, BK), lambda kb, nb: (nb, kb)),
            pl.BlockSpec((1, 128), lambda kb, nb: (0, 0)),
        ],
        out_shape=[
            jax.ShapeDtypeStruct((N, K), jnp.float32),
            jax.ShapeDtypeStruct((1, 128), jnp.float32),
        ],
        scratch_shapes=[
            pltpu.VMEM((1, BK), jnp.float32),
            pltpu.SMEM((1, 1), jnp.float32),
        ],
    )(idx_flat)


def kernel(inputTensor, emb_weights, ema_w, ema_cluster_size):
    del ema_w, ema_cluster_size
    flat = inputTensor.reshape(-1, D)
    xt = flat.T

    idx_flat, loss_out = _argmin_call(xt, emb_weights)

    sc_gather = functools.partial(
        pl.kernel,
        mesh=plsc.VectorSubcoreMesh(core_axis_name="c", subcore_axis_name="s"),
        out_type=jax.ShapeDtypeStruct((N, D), jnp.float32),
        scratch_types=[
            pltpu.VMEM((_B_PER_W,), jnp.int32),
            pltpu.VMEM((_B_PER_W, D), jnp.float32),
            pltpu.SemaphoreType.DMA,
        ],
    )(_sc_gather)
    quantized = sc_gather(emb_weights, idx_flat)

    enc, perp_out = _onehot_call(idx_flat)

    loss = loss_out[0, 0]
    perplexity = perp_out[0, 0]
    quantized_st = quantized.reshape(inputTensor.shape)
    return (loss, quantized_st, perplexity, enc)


# speedup vs baseline: 1.2280x; 1.1097x over previous
import functools

import jax
import jax.numpy as jnp
from jax import lax
from jax.experimental import pallas as pl
from jax.experimental.pallas import tpu as pltpu
from jax.experimental.pallas import tpu_sc as plsc

D = 256
K = 8192
N = 8192
NB = 2
KB = 8
BN = N // NB
BK = K // KB

_SC_NC = 2
_SC_NS = 16
_SC_NW = _SC_NC * _SC_NS
_B_PER_W = N // _SC_NW


def _argmin_body(xt_ref, w_ref, idx_ref, loss_ref, mv_s, mi_s, acc_s):
    nb = pl.program_id(0)
    kb = pl.program_id(1)
    xt = xt_ref[...]
    w = w_ref[...]
    w2 = w * jnp.float32(-2.0)
    s = lax.dot_general(w2, xt, (((1,), (0,)), ((), ())),
                        preferred_element_type=jnp.float32)
    s1 = jnp.sum(xt * xt, axis=0, keepdims=True)
    s2 = jnp.sum(w * w, axis=1, keepdims=True)
    d = (s1 + s) + s2
    bmin = jnp.min(d, axis=0, keepdims=True)
    bidx = (jnp.argmin(d, axis=0).astype(jnp.float32).reshape(1, BN)
            + (kb * BK).astype(jnp.float32))

    @pl.when(kb == 0)
    def _():
        mv_s[...] = bmin
        mi_s[...] = bidx

    @pl.when(kb > 0)
    def _():
        better = bmin < mv_s[...]
        mi_s[...] = jnp.where(better, bidx, mi_s[...])
        mv_s[...] = jnp.where(better, bmin, mv_s[...])

    @pl.when(kb == KB - 1)
    def _():
        idx_ref[...] = mi_s[...].astype(jnp.int32).reshape(BN)
        rowsum = jnp.sum(mv_s[...])

        @pl.when(nb == 0)
        def _():
            acc_s[0, 0] = rowsum

        @pl.when(nb > 0)
        def _():
            acc_s[0, 0] = acc_s[0, 0] + rowsum

        @pl.when(nb == NB - 1)
        def _():
            loss = 0.25 * acc_s[0, 0] * (1.0 / (N * D))
            loss_ref[...] = jnp.full((1, 128), loss, jnp.float32)


def _onehot_body(idx_ref, enc_ref, perp_ref, cnt_s, ent_s):
    kb = pl.program_id(0)
    nb = pl.program_id(1)
    idx_row = idx_ref[...].reshape(1, BN)
    idx_col = lax.transpose(idx_row, (1, 0))
    ids = lax.broadcasted_iota(jnp.int32, (BN, BK), 1) + kb * BK
    onehot = (ids == idx_col).astype(jnp.float32)
    enc_ref[...] = onehot
    cnt = jnp.sum(onehot, axis=0, keepdims=True)

    @pl.when(nb == 0)
    def _():
        cnt_s[...] = cnt

    @pl.when(nb > 0)
    def _():
        cnt_s[...] = cnt_s[...] + cnt

    @pl.when(nb == NB - 1)
    def _():
        p = cnt_s[...] * (1.0 / N)
        ev = jnp.sum(p * jnp.log(p + 1e-10))

        @pl.when(kb == 0)
        def _():
            ent_s[0, 0] = ev

        @pl.when(kb > 0)
        def _():
            ent_s[0, 0] = ent_s[0, 0] + ev

        @pl.when(kb == KB - 1)
        def _():
            perp_ref[...] = jnp.exp(jnp.full((1, 128), -ent_s[0, 0],
                                             jnp.float32))


def _sc_gather(table_hbm, idx_hbm, out_hbm, idx_v, rows_v, sem):
    wid = lax.axis_index("s") * _SC_NC + lax.axis_index("c")
    base = wid * _B_PER_W
    pltpu.sync_copy(idx_hbm.at[pl.ds(base, _B_PER_W)], idx_v)
    pltpu.async_copy(table_hbm.at[idx_v], rows_v, sem).wait()
    pltpu.sync_copy(rows_v, out_hbm.at[pl.ds(base, _B_PER_W)])


def _argmin_call(xt, w):
    return pl.pallas_call(
        _argmin_body,
        grid=(NB, KB),
        in_specs=[
            pl.BlockSpec((D, BN), lambda nb, kb: (0, nb)),
            pl.BlockSpec((BK, D), lambda nb, kb: (kb, 0)),
        ],
        out_specs=[
            pl.BlockSpec((BN,), lambda nb, kb: (nb,)),
            pl.BlockSpec((1, 128), lambda nb, kb: (0, 0)),
        ],
        out_shape=[
            jax.ShapeDtypeStruct((N,), jnp.int32),
            jax.ShapeDtypeStruct((1, 128), jnp.float32),
        ],
        scratch_shapes=[
            pltpu.VMEM((1, BN), jnp.float32),
            pltpu.VMEM((1, BN), jnp.float32),
            pltpu.SMEM((1, 1), jnp.float32),
        ],
    )(xt, w)


def _onehot_call(idx_flat):
    return pl.pallas_call(
        _onehot_body,
        grid=(KB, NB),
        in_specs=[
            pl.BlockSpec((BN,), lambda kb, nb: (nb,)),
        ],
        out_specs=[
            pl.BlockSpec((BN, BK), lambda kb, nb: (nb, kb)),
            pl.BlockSpec((1, 128), lambda kb, nb: (0, 0)),
        ],
        out_shape=[
            jax.ShapeDtypeStruct((N, K), jnp.float32),
            jax.ShapeDtypeStruct((1, 128), jnp.float32),
        ],
        scratch_shapes=[
            pltpu.VMEM((1, BK), jnp.float32),
            pltpu.SMEM((1, 1), jnp.float32),
        ],
    )(idx_flat)


def kernel(inputTensor, emb_weights, ema_w, ema_cluster_size):
    del ema_w, ema_cluster_size
    flat = inputTensor.reshape(-1, D)
    xt = flat.T

    idx_flat, loss_out = _argmin_call(xt, emb_weights)

    sc_gather = functools.partial(
        pl.kernel,
        mesh=plsc.VectorSubcoreMesh(core_axis_name="c", subcore_axis_name="s"),
        out_type=jax.ShapeDtypeStruct((N, D), jnp.float32),
        scratch_types=[
            pltpu.VMEM((_B_PER_W,), jnp.int32),
            pltpu.VMEM((_B_PER_W, D), jnp.float32),
            pltpu.SemaphoreType.DMA,
        ],
    )(_sc_gather)
    quantized = sc_gather(emb_weights, idx_flat)

    enc, perp_out = _onehot_call(idx_flat)

    loss = loss_out[0, 0]
    perplexity = perp_out[0, 0]
    quantized_st = quantized.reshape(inputTensor.shape)
    return (loss, quantized_st, perplexity, enc)
